# baseline (speedup 1.0000x reference)
import jax
import jax.numpy as jnp
import numpy as np
from jax import lax
from jax.experimental import pallas as pl
from jax.experimental.pallas import tpu as pltpu
from jax.experimental.pallas import tpu_sc as plsc

B = 8
N = 4096
S = 512
K = 32
L = 16
NC = 2
NS = 16
NW = NC * NS
QPW = (B * S) // NW
QBLK = S // QPW
CHUNKS = N // L
UNROLL = 16
G = 2
RSQ = np.float32(0.2 * 0.2)


def _ball_query_body(posx_hbm, cents_hbm, out_hbm,
                     x_v, y_v, z_v, pp_v, cid_v,
                     cxs_v, cys_v, czs_v, ccs_v, nbr0_v, nbr1_v, nbr2_v, nbr3_v, outb_v):
    cid = lax.axis_index("c")
    sid = lax.axis_index("s")
    wid = sid * NC + cid
    b = wid // QBLK
    qbase = (wid % QBLK) * QPW

    pltpu.sync_copy(posx_hbm.at[pl.ds((b * 3 + 0) * N, N)], x_v)
    pltpu.sync_copy(posx_hbm.at[pl.ds((b * 3 + 1) * N, N)], y_v)
    pltpu.sync_copy(posx_hbm.at[pl.ds((b * 3 + 2) * N, N)], z_v)
    pltpu.sync_copy(cents_hbm.at[pl.ds(b * S + qbase, QPW)], cid_v)

    def bf16r(v):
        bi = lax.bitcast_convert_type(v, jnp.int32)
        r = bi + jnp.int32(0x7FFF) + ((bi >> 16) & 1)
        return lax.bitcast_convert_type(r & jnp.int32(-65536), jnp.float32)

    def center_step(i, _):
        s = pl.ds(i * L, L)
        idxv = cid_v[s]
        cxv = plsc.load_gather(x_v, [idxv])
        cyv = plsc.load_gather(y_v, [idxv])
        czv = plsc.load_gather(z_v, [idxv])
        ccs_v[s] = (cxv * cxv + cyv * cyv) + czv * czv
        n2 = np.float32(-2.0)
        cxs_v[s] = bf16r(cxv) * n2
        cys_v[s] = bf16r(cyv) * n2
        czs_v[s] = bf16r(czv) * n2
        return 0

    lax.fori_loop(0, QPW // L, center_step, 0)

    def pp_step(i, _):
        s = pl.ds(i * L, L)
        xv = x_v[s]
        yv = y_v[s]
        zv = z_v[s]
        pp_v[s] = (xv * xv + yv * yv) + zv * zv
        x_v[s] = bf16r(xv)
        y_v[s] = bf16r(yv)
        z_v[s] = bf16r(zv)
        return 0

    lax.fori_loop(0, CHUNKS, pp_step, 0)

    lanes = lax.broadcasted_iota(jnp.int32, (L,), 0)
    nbrs = [nbr0_v, nbr1_v, nbr2_v, nbr3_v][:G]

    def one_group(i, _):
        qs = [i + g * (QPW // G) for g in range(G)]
        cxs = [cxs_v[pl.ds(q, L)][0] for q in qs]
        cys = [cys_v[pl.ds(q, L)][0] for q in qs]
        czs = [czs_v[pl.ds(q, L)][0] for q in qs]
        ccs = [ccs_v[pl.ds(q, L)][0] for q in qs]

        def cond(carry):
            chunk = carry[0]
            unfinished = carry[1] < K
            for g in range(2, G + 1):
                unfinished = jnp.logical_or(unfinished, carry[g] < K)
            return jnp.logical_and(chunk < CHUNKS, unfinished)

        def body(carry):
            chunk = carry[0]
            cnts = list(carry[1:])
            base = chunk * L
            xs, ys, zs, ps, idxs = [], [], [], [], []
            for u in range(UNROLL):
                s = pl.ds(base + u * L, L)
                xs.append(x_v[s])
                ys.append(y_v[s])
                zs.append(z_v[s])
                ps.append(pp_v[s])
                idxs.append(lanes + (base + u * L))
            ms = [[None] * UNROLL for _ in range(G)]
            for u in range(UNROLL):
                for g in range(G):
                    t = (cxs[g] * xs[u] + cys[g] * ys[u]) + czs[g] * zs[u]
                    d = (t + ccs[g]) + ps[u]
                    ms[g][u] = d <= RSQ
            cs = [[plsc.all_reduce_population_count(m)[0] for m in ms[g]]
                  for g in range(G)]
            for g in range(G):
                for u in range(UNROLL):
                    plsc.store_compressed(nbrs[g].at[pl.ds(cnts[g], L)],
                                          idxs[u], mask=ms[g][u])
                    cnts[g] = cnts[g] + cs[g][u]
            return (chunk + UNROLL, *cnts)

        carry = lax.while_loop(cond, body, (jnp.int32(0),) + (jnp.int32(0),) * G)
        cnts = carry[1:]

        for g in range(G):
            q = qs[g]
            v0 = nbrs[g][pl.ds(0, L)]
            v1 = nbrs[g][pl.ds(L, L)]
            first = v0[0]
            outb_v[pl.ds(q * K, L)] = jnp.where(lanes < cnts[g], v0, first)
            outb_v[pl.ds(q * K + L, L)] = jnp.where(lanes + L < cnts[g], v1, first)
        return 0

    lax.fori_loop(0, QPW // G, one_group, 0)

    pltpu.sync_copy(outb_v, out_hbm.at[pl.ds((b * S + qbase) * K, QPW * K)])


@jax.jit
def _ball_query(posx, centroids):
    mesh = plsc.VectorSubcoreMesh(core_axis_name="c", subcore_axis_name="s")
    run = pl.kernel(
        _ball_query_body,
        out_type=jax.ShapeDtypeStruct((B * S * K,), jnp.int32),
        mesh=mesh,
        compiler_params=pltpu.CompilerParams(needs_layout_passes=False),
        scratch_types=[
            pltpu.VMEM((N,), jnp.float32),
            pltpu.VMEM((N,), jnp.float32),
            pltpu.VMEM((N,), jnp.float32),
            pltpu.VMEM((N,), jnp.float32),
            pltpu.VMEM((QPW,), jnp.int32),
            pltpu.VMEM((QPW + L,), jnp.float32),
            pltpu.VMEM((QPW + L,), jnp.float32),
            pltpu.VMEM((QPW + L,), jnp.float32),
            pltpu.VMEM((QPW + L,), jnp.float32),
            pltpu.VMEM((N + UNROLL * L,), jnp.int32),
            pltpu.VMEM((N + UNROLL * L,), jnp.int32),
            pltpu.VMEM((N + UNROLL * L,), jnp.int32),
            pltpu.VMEM((N + UNROLL * L,), jnp.int32),
            pltpu.VMEM((QPW * K,), jnp.int32),
        ],
    )
    return run(posx, centroids).reshape(B, S, K)


def kernel(pos, centroids, centroids_index, index_voxels):
    del centroids_index, index_voxels
    posx = jnp.transpose(pos, (0, 2, 1)).reshape(-1)
    return _ball_query(posx, centroids.reshape(-1))


# speedup vs baseline: 1.1843x; 1.1843x over previous
import jax
import jax.numpy as jnp
import numpy as np
from jax import lax
from jax.experimental import pallas as pl
from jax.experimental.pallas import tpu as pltpu
from jax.experimental.pallas import tpu_sc as plsc

B = 8
N = 4096
S = 512
K = 32
L = 16
NC = 2
NS = 16
NW = NC * NS
QPW = (B * S) // NW
QBLK = S // QPW
CHUNKS = N // L
UNROLL = 8
G = 2
RSQ = np.float32(0.2 * 0.2)


def _ball_query_body(posx_hbm, cents_hbm, out_hbm,
                     x_v, y_v, z_v, pp_v, cid_v,
                     cxs_v, cys_v, czs_v, ccs_v, nbr0_v, nbr1_v, nbr2_v, nbr3_v, outb_v):
    cid = lax.axis_index("c")
    sid = lax.axis_index("s")
    wid = sid * NC + cid
    b = wid // QBLK
    qbase = (wid % QBLK) * QPW

    pltpu.sync_copy(posx_hbm.at[pl.ds((b * 3 + 0) * N, N)], x_v.at[pl.ds(0, N)])
    pltpu.sync_copy(posx_hbm.at[pl.ds((b * 3 + 1) * N, N)], y_v.at[pl.ds(0, N)])
    pltpu.sync_copy(posx_hbm.at[pl.ds((b * 3 + 2) * N, N)], z_v.at[pl.ds(0, N)])
    pltpu.sync_copy(cents_hbm.at[pl.ds(b * S + qbase, QPW)], cid_v)

    def bf16r(v):
        bi = lax.bitcast_convert_type(v, jnp.int32)
        r = bi + jnp.int32(0x7FFF) + ((bi >> 16) & 1)
        return lax.bitcast_convert_type(r & jnp.int32(-65536), jnp.float32)

    def center_step(i, _):
        s = pl.ds(i * L, L)
        idxv = cid_v[s]
        cxv = plsc.load_gather(x_v, [idxv])
        cyv = plsc.load_gather(y_v, [idxv])
        czv = plsc.load_gather(z_v, [idxv])
        ccs_v[s] = (cxv * cxv + cyv * cyv) + czv * czv
        n2 = np.float32(-2.0)
        cxs_v[s] = bf16r(cxv) * n2
        cys_v[s] = bf16r(cyv) * n2
        czs_v[s] = bf16r(czv) * n2
        return 0

    lax.fori_loop(0, QPW // L, center_step, 0)

    def pp_step(i, _):
        s = pl.ds(i * L, L)
        xv = x_v[s]
        yv = y_v[s]
        zv = z_v[s]
        pp_v[s] = (xv * xv + yv * yv) + zv * zv
        x_v[s] = bf16r(xv)
        y_v[s] = bf16r(yv)
        z_v[s] = bf16r(zv)
        return 0

    lax.fori_loop(0, CHUNKS, pp_step, 0)

    lanes = lax.broadcasted_iota(jnp.int32, (L,), 0)
    nbrs = [nbr0_v, nbr1_v, nbr2_v, nbr3_v][:G]

    def one_group(i, _):
        qs = [i + g * (QPW // G) for g in range(G)]
        cxs = [cxs_v[pl.ds(q, L)][0] for q in qs]
        cys = [cys_v[pl.ds(q, L)][0] for q in qs]
        czs = [czs_v[pl.ds(q, L)][0] for q in qs]
        ccs = [ccs_v[pl.ds(q, L)][0] for q in qs]

        def load_chunks(base):
            vecs = []
            for u in range(UNROLL):
                s = pl.ds(base + u * L, L)
                vecs.extend((x_v[s], y_v[s], z_v[s], pp_v[s]))
            return tuple(vecs)

        def cond(carry):
            chunk = carry[0]
            unfinished = carry[1] < K
            for g in range(2, G + 1):
                unfinished = jnp.logical_or(unfinished, carry[g] < K)
            return jnp.logical_and(chunk < CHUNKS, unfinished)

        def body(carry):
            chunk = carry[0]
            cnts = list(carry[1:1 + G])
            vecs = carry[1 + G:]
            base = chunk * L
            nxt = load_chunks(base + UNROLL * L)
            idxs = [lanes + (base + u * L) for u in range(UNROLL)]
            ms = [[None] * UNROLL for _ in range(G)]
            for u in range(UNROLL):
                xu, yu, zu, pu = vecs[4 * u:4 * u + 4]
                for g in range(G):
                    t = (cxs[g] * xu + cys[g] * yu) + czs[g] * zu
                    d = (t + ccs[g]) + pu
                    ms[g][u] = d <= RSQ
            cs = [[plsc.all_reduce_population_count(m)[0] for m in ms[g]]
                  for g in range(G)]
            for g in range(G):
                for u in range(UNROLL):
                    plsc.store_compressed(nbrs[g].at[pl.ds(cnts[g], L)],
                                          idxs[u], mask=ms[g][u])
                    cnts[g] = cnts[g] + cs[g][u]
            return (chunk + UNROLL, *cnts, *nxt)

        carry = lax.while_loop(
            cond, body,
            (jnp.int32(0),) + (jnp.int32(0),) * G + load_chunks(0))
        cnts = carry[1:1 + G]

        for g in range(G):
            q = qs[g]
            v0 = nbrs[g][pl.ds(0, L)]
            v1 = nbrs[g][pl.ds(L, L)]
            first = v0[0]
            outb_v[pl.ds(q * K, L)] = jnp.where(lanes < cnts[g], v0, first)
            outb_v[pl.ds(q * K + L, L)] = jnp.where(lanes + L < cnts[g], v1, first)
        return 0

    lax.fori_loop(0, QPW // G, one_group, 0)

    pltpu.sync_copy(outb_v, out_hbm.at[pl.ds((b * S + qbase) * K, QPW * K)])


@jax.jit
def _ball_query(posx, centroids):
    mesh = plsc.VectorSubcoreMesh(core_axis_name="c", subcore_axis_name="s")
    run = pl.kernel(
        _ball_query_body,
        out_type=jax.ShapeDtypeStruct((B * S * K,), jnp.int32),
        mesh=mesh,
        compiler_params=pltpu.CompilerParams(needs_layout_passes=False),
        scratch_types=[
            pltpu.VMEM((N + UNROLL * L,), jnp.float32),
            pltpu.VMEM((N + UNROLL * L,), jnp.float32),
            pltpu.VMEM((N + UNROLL * L,), jnp.float32),
            pltpu.VMEM((N + UNROLL * L,), jnp.float32),
            pltpu.VMEM((QPW,), jnp.int32),
            pltpu.VMEM((QPW + L,), jnp.float32),
            pltpu.VMEM((QPW + L,), jnp.float32),
            pltpu.VMEM((QPW + L,), jnp.float32),
            pltpu.VMEM((QPW + L,), jnp.float32),
            pltpu.VMEM((N + UNROLL * L,), jnp.int32),
            pltpu.VMEM((N + UNROLL * L,), jnp.int32),
            pltpu.VMEM((N + UNROLL * L,), jnp.int32),
            pltpu.VMEM((N + UNROLL * L,), jnp.int32),
            pltpu.VMEM((QPW * K,), jnp.int32),
        ],
    )
    return run(posx, centroids).reshape(B, S, K)


def kernel(pos, centroids, centroids_index, index_voxels):
    del centroids_index, index_voxels
    posx = jnp.transpose(pos, (0, 2, 1)).reshape(-1)
    return _ball_query(posx, centroids.reshape(-1))


# speedup vs baseline: 1.2542x; 1.0590x over previous
import jax
import jax.numpy as jnp
import numpy as np
from jax import lax
from jax.experimental import pallas as pl
from jax.experimental.pallas import tpu as pltpu
from jax.experimental.pallas import tpu_sc as plsc

B = 8
N = 4096
S = 512
K = 32
L = 16
NC = 2
NS = 16
NW = NC * NS
QPW = (B * S) // NW
QBLK = S // QPW
CHUNKS = N // L
UNROLL = 4
G = 2
RSQ = np.float32(0.2 * 0.2)


def _ball_query_body(posx_hbm, cents_hbm, out_hbm,
                     x_v, y_v, z_v, pp_v, cid_v,
                     cxs_v, cys_v, czs_v, ccs_v, nbr0_v, nbr1_v, nbr2_v, nbr3_v, outb_v):
    cid = lax.axis_index("c")
    sid = lax.axis_index("s")
    wid = sid * NC + cid
    b = wid // QBLK
    qbase = (wid % QBLK) * QPW

    pltpu.sync_copy(posx_hbm.at[pl.ds((b * 3 + 0) * N, N)], x_v.at[pl.ds(0, N)])
    pltpu.sync_copy(posx_hbm.at[pl.ds((b * 3 + 1) * N, N)], y_v.at[pl.ds(0, N)])
    pltpu.sync_copy(posx_hbm.at[pl.ds((b * 3 + 2) * N, N)], z_v.at[pl.ds(0, N)])
    pltpu.sync_copy(cents_hbm.at[pl.ds(b * S + qbase, QPW)], cid_v)

    def bf16r(v):
        bi = lax.bitcast_convert_type(v, jnp.int32)
        r = bi + jnp.int32(0x7FFF) + ((bi >> 16) & 1)
        return lax.bitcast_convert_type(r & jnp.int32(-65536), jnp.float32)

    def center_step(i, _):
        s = pl.ds(i * L, L)
        idxv = cid_v[s]
        cxv = plsc.load_gather(x_v, [idxv])
        cyv = plsc.load_gather(y_v, [idxv])
        czv = plsc.load_gather(z_v, [idxv])
        ccs_v[s] = (cxv * cxv + cyv * cyv) + czv * czv
        n2 = np.float32(-2.0)
        cxs_v[s] = bf16r(cxv) * n2
        cys_v[s] = bf16r(cyv) * n2
        czs_v[s] = bf16r(czv) * n2
        return 0

    lax.fori_loop(0, QPW // L, center_step, 0)

    def pp_step(i, _):
        s = pl.ds(i * L, L)
        xv = x_v[s]
        yv = y_v[s]
        zv = z_v[s]
        pp_v[s] = (xv * xv + yv * yv) + zv * zv
        x_v[s] = bf16r(xv)
        y_v[s] = bf16r(yv)
        z_v[s] = bf16r(zv)
        return 0

    lax.fori_loop(0, CHUNKS, pp_step, 0)

    lanes = lax.broadcasted_iota(jnp.int32, (L,), 0)
    nbrs = [nbr0_v, nbr1_v, nbr2_v, nbr3_v][:G]

    def one_group(i, _):
        qs = [i + g * (QPW // G) for g in range(G)]
        cxs = [cxs_v[pl.ds(q, L)][0] for q in qs]
        cys = [cys_v[pl.ds(q, L)][0] for q in qs]
        czs = [czs_v[pl.ds(q, L)][0] for q in qs]
        ccs = [ccs_v[pl.ds(q, L)][0] for q in qs]

        def load_chunks(base):
            vecs = []
            for u in range(UNROLL):
                s = pl.ds(base + u * L, L)
                vecs.extend((x_v[s], y_v[s], z_v[s], pp_v[s]))
            return tuple(vecs)

        def compute_ds(vecs):
            ds = []
            for u in range(UNROLL):
                xu, yu, zu, pu = vecs[4 * u:4 * u + 4]
                for g in range(G):
                    t = (cxs[g] * xu + cys[g] * yu) + czs[g] * zu
                    ds.append((t + ccs[g]) + pu)
            return ds

        def store_ds(ds, base, cnts):
            ms = [[ds[u * G + g] <= RSQ for u in range(UNROLL)]
                  for g in range(G)]
            idxs = [lanes + (base + u * L) for u in range(UNROLL)]
            cs = [[plsc.all_reduce_population_count(m)[0] for m in ms[g]]
                  for g in range(G)]
            cnts = list(cnts)
            for g in range(G):
                for u in range(UNROLL):
                    plsc.store_compressed(nbrs[g].at[pl.ds(cnts[g], L)],
                                          idxs[u], mask=ms[g][u])
                    cnts[g] = cnts[g] + cs[g][u]
            return cnts

        NV = 4 * UNROLL
        ND = G * UNROLL

        def cond(carry):
            chunk = carry[0]
            unfinished = carry[1] < K
            for g in range(2, G + 1):
                unfinished = jnp.logical_or(unfinished, carry[g] < K)
            return jnp.logical_and(chunk < CHUNKS, unfinished)

        def body(carry):
            chunk = carry[0]
            cnts = list(carry[1:1 + G])
            vecs = carry[1 + G:1 + G + NV]
            pend = carry[1 + G + NV:]
            base = chunk * L
            nxt = load_chunks(base + UNROLL * L)
            ds = compute_ds(vecs)
            cnts = store_ds(pend, base - UNROLL * L, cnts)
            return (chunk + UNROLL, *cnts, *nxt, *ds)

        inf = jnp.full((L,), jnp.inf, jnp.float32)
        carry = lax.while_loop(
            cond, body,
            (jnp.int32(0),) + (jnp.int32(0),) * G + load_chunks(0)
            + (inf,) * ND)
        final_chunk = carry[0]
        cnts = store_ds(carry[1 + G + NV:], final_chunk * L - UNROLL * L,
                        carry[1:1 + G])

        for g in range(G):
            q = qs[g]
            v0 = nbrs[g][pl.ds(0, L)]
            v1 = nbrs[g][pl.ds(L, L)]
            first = v0[0]
            outb_v[pl.ds(q * K, L)] = jnp.where(lanes < cnts[g], v0, first)
            outb_v[pl.ds(q * K + L, L)] = jnp.where(lanes + L < cnts[g], v1, first)
        return 0

    lax.fori_loop(0, QPW // G, one_group, 0)

    pltpu.sync_copy(outb_v, out_hbm.at[pl.ds((b * S + qbase) * K, QPW * K)])


@jax.jit
def _ball_query(posx, centroids):
    mesh = plsc.VectorSubcoreMesh(core_axis_name="c", subcore_axis_name="s")
    run = pl.kernel(
        _ball_query_body,
        out_type=jax.ShapeDtypeStruct((B * S * K,), jnp.int32),
        mesh=mesh,
        compiler_params=pltpu.CompilerParams(needs_layout_passes=False),
        scratch_types=[
            pltpu.VMEM((N + UNROLL * L,), jnp.float32),
            pltpu.VMEM((N + UNROLL * L,), jnp.float32),
            pltpu.VMEM((N + UNROLL * L,), jnp.float32),
            pltpu.VMEM((N + UNROLL * L,), jnp.float32),
            pltpu.VMEM((QPW,), jnp.int32),
            pltpu.VMEM((QPW + L,), jnp.float32),
            pltpu.VMEM((QPW + L,), jnp.float32),
            pltpu.VMEM((QPW + L,), jnp.float32),
            pltpu.VMEM((QPW + L,), jnp.float32),
            pltpu.VMEM((N + UNROLL * L,), jnp.int32),
            pltpu.VMEM((N + UNROLL * L,), jnp.int32),
            pltpu.VMEM((N + UNROLL * L,), jnp.int32),
            pltpu.VMEM((N + UNROLL * L,), jnp.int32),
            pltpu.VMEM((QPW * K,), jnp.int32),
        ],
    )
    return run(posx, centroids).reshape(B, S, K)


def kernel(pos, centroids, centroids_index, index_voxels):
    del centroids_index, index_voxels
    posx = jnp.transpose(pos, (0, 2, 1)).reshape(-1)
    return _ball_query(posx, centroids.reshape(-1))


# speedup vs baseline: 1.2582x; 1.0031x over previous
import jax
import jax.numpy as jnp
import numpy as np
from jax import lax
from jax.experimental import pallas as pl
from jax.experimental.pallas import tpu as pltpu
from jax.experimental.pallas import tpu_sc as plsc

B = 8
N = 4096
S = 512
K = 32
L = 16
NC = 2
NS = 16
NW = NC * NS
QPW = (B * S) // NW
QBLK = S // QPW
CHUNKS = N // L
UNROLL = 8
G = 2
RSQ = np.float32(0.2 * 0.2)


def _ball_query_body(posx_hbm, cents_hbm, out_hbm,
                     x_v, y_v, z_v, pp_v, cid_v,
                     cxs_v, cys_v, czs_v, ccs_v, nbr0_v, nbr1_v, nbr2_v, nbr3_v, outb_v):
    cid = lax.axis_index("c")
    sid = lax.axis_index("s")
    wid = sid * NC + cid
    b = wid // QBLK
    qbase = (wid % QBLK) * QPW

    pltpu.sync_copy(posx_hbm.at[pl.ds((b * 3 + 0) * N, N)], x_v.at[pl.ds(0, N)])
    pltpu.sync_copy(posx_hbm.at[pl.ds((b * 3 + 1) * N, N)], y_v.at[pl.ds(0, N)])
    pltpu.sync_copy(posx_hbm.at[pl.ds((b * 3 + 2) * N, N)], z_v.at[pl.ds(0, N)])
    pltpu.sync_copy(cents_hbm.at[pl.ds(b * S + qbase, QPW)], cid_v)

    def bf16r(v):
        bi = lax.bitcast_convert_type(v, jnp.int32)
        r = bi + jnp.int32(0x7FFF) + ((bi >> 16) & 1)
        return lax.bitcast_convert_type(r & jnp.int32(-65536), jnp.float32)

    def center_step(i, _):
        s = pl.ds(i * L, L)
        idxv = cid_v[s]
        cxv = plsc.load_gather(x_v, [idxv])
        cyv = plsc.load_gather(y_v, [idxv])
        czv = plsc.load_gather(z_v, [idxv])
        ccs_v[s] = (cxv * cxv + cyv * cyv) + czv * czv
        n2 = np.float32(-2.0)
        cxs_v[s] = bf16r(cxv) * n2
        cys_v[s] = bf16r(cyv) * n2
        czs_v[s] = bf16r(czv) * n2
        return 0

    lax.fori_loop(0, QPW // L, center_step, 0)

    def pp_step(i, _):
        s = pl.ds(i * L, L)
        xv = x_v[s]
        yv = y_v[s]
        zv = z_v[s]
        pp_v[s] = (xv * xv + yv * yv) + zv * zv
        x_v[s] = bf16r(xv)
        y_v[s] = bf16r(yv)
        z_v[s] = bf16r(zv)
        return 0

    lax.fori_loop(0, CHUNKS, pp_step, 0)

    lanes = lax.broadcasted_iota(jnp.int32, (L,), 0)
    nbrs = [nbr0_v, nbr1_v, nbr2_v, nbr3_v][:G]

    def one_group(i, _):
        qs = [i + g * (QPW // G) for g in range(G)]
        cxs = [cxs_v[pl.ds(q, L)][0] for q in qs]
        cys = [cys_v[pl.ds(q, L)][0] for q in qs]
        czs = [czs_v[pl.ds(q, L)][0] for q in qs]
        ccs = [ccs_v[pl.ds(q, L)][0] for q in qs]

        def load_chunks(base):
            vecs = []
            for u in range(UNROLL):
                s = pl.ds(base + u * L, L)
                vecs.extend((x_v[s], y_v[s], z_v[s], pp_v[s]))
            return tuple(vecs)

        def compute_ds(vecs):
            ds = []
            for u in range(UNROLL):
                xu, yu, zu, pu = vecs[4 * u:4 * u + 4]
                for g in range(G):
                    t = (cxs[g] * xu + cys[g] * yu) + czs[g] * zu
                    ds.append((t + ccs[g]) + pu)
            return ds

        def store_ds(ds, base, cnts):
            ms = [[ds[u * G + g] <= RSQ for u in range(UNROLL)]
                  for g in range(G)]
            idxs = [lanes + (base + u * L) for u in range(UNROLL)]
            cs = [[plsc.all_reduce_population_count(m)[0] for m in ms[g]]
                  for g in range(G)]
            cnts = list(cnts)
            for g in range(G):
                for u in range(UNROLL):
                    plsc.store_compressed(nbrs[g].at[pl.ds(cnts[g], L)],
                                          idxs[u], mask=ms[g][u])
                    cnts[g] = cnts[g] + cs[g][u]
            return cnts

        NV = 4 * UNROLL
        ND = G * UNROLL

        def cond(carry):
            chunk = carry[0]
            unfinished = carry[1] < K
            for g in range(2, G + 1):
                unfinished = jnp.logical_or(unfinished, carry[g] < K)
            return jnp.logical_and(chunk < CHUNKS, unfinished)

        def body(carry):
            chunk = carry[0]
            cnts = list(carry[1:1 + G])
            vecs = carry[1 + G:1 + G + NV]
            pend = carry[1 + G + NV:]
            base = chunk * L
            nxt = load_chunks(base + UNROLL * L)
            ds = compute_ds(vecs)
            cnts = store_ds(pend, base - UNROLL * L, cnts)
            return (chunk + UNROLL, *cnts, *nxt, *ds)

        inf = jnp.full((L,), jnp.inf, jnp.float32)
        carry = lax.while_loop(
            cond, body,
            (jnp.int32(0),) + (jnp.int32(0),) * G + load_chunks(0)
            + (inf,) * ND)
        final_chunk = carry[0]
        cnts = store_ds(carry[1 + G + NV:], final_chunk * L - UNROLL * L,
                        carry[1:1 + G])

        for g in range(G):
            q = qs[g]
            v0 = nbrs[g][pl.ds(0, L)]
            v1 = nbrs[g][pl.ds(L, L)]
            first = v0[0]
            outb_v[pl.ds(q * K, L)] = jnp.where(lanes < cnts[g], v0, first)
            outb_v[pl.ds(q * K + L, L)] = jnp.where(lanes + L < cnts[g], v1, first)
        return 0

    lax.fori_loop(0, QPW // G, one_group, 0)

    pltpu.sync_copy(outb_v, out_hbm.at[pl.ds((b * S + qbase) * K, QPW * K)])


@jax.jit
def _ball_query(posx, centroids):
    mesh = plsc.VectorSubcoreMesh(core_axis_name="c", subcore_axis_name="s")
    run = pl.kernel(
        _ball_query_body,
        out_type=jax.ShapeDtypeStruct((B * S * K,), jnp.int32),
        mesh=mesh,
        compiler_params=pltpu.CompilerParams(needs_layout_passes=False),
        scratch_types=[
            pltpu.VMEM((N + UNROLL * L,), jnp.float32),
            pltpu.VMEM((N + UNROLL * L,), jnp.float32),
            pltpu.VMEM((N + UNROLL * L,), jnp.float32),
            pltpu.VMEM((N + UNROLL * L,), jnp.float32),
            pltpu.VMEM((QPW,), jnp.int32),
            pltpu.VMEM((QPW + L,), jnp.float32),
            pltpu.VMEM((QPW + L,), jnp.float32),
            pltpu.VMEM((QPW + L,), jnp.float32),
            pltpu.VMEM((QPW + L,), jnp.float32),
            pltpu.VMEM((N + UNROLL * L,), jnp.int32),
            pltpu.VMEM((N + UNROLL * L,), jnp.int32),
            pltpu.VMEM((N + UNROLL * L,), jnp.int32),
            pltpu.VMEM((N + UNROLL * L,), jnp.int32),
            pltpu.VMEM((QPW * K,), jnp.int32),
        ],
    )
    return run(posx, centroids).reshape(B, S, K)


def kernel(pos, centroids, centroids_index, index_voxels):
    del centroids_index, index_voxels
    posx = jnp.transpose(pos, (0, 2, 1)).reshape(-1)
    return _ball_query(posx, centroids.reshape(-1))
